# Initial kernel scaffold; baseline (speedup 1.0000x reference)
import jax
import jax.numpy as jnp
from jax.experimental import pallas as pl


def kernel(x, edge_index, batch, W_gcn1, b_gcn1, W_gcn2, b_gcn2, W_gcn3, b_gcn3, W_gat, a_src, a_dst, b_gat, W_c1, b_c1, W_c2, b_c2, W_c3, b_c3):
    raise NotImplementedError("write your pallas kernel here")


# speedup vs baseline: 15.2352x; 15.2352x over previous
import functools
import jax
import jax.numpy as jnp
from jax import lax
from jax.experimental import pallas as pl
from jax.experimental.pallas import tpu as pltpu
from jax.experimental.pallas import tpu_sc as plsc

N = 10000
E = 320000
HID = 128
NHEAD = 4
GDIM = NHEAD * HID
NG = 64
NC = 2
NS = 16
NW = NC * NS
EW = E // NW
CH = 80
NCH = EW // CH
NGRP = CH // 16
SLAB_STEP = 624
SLAB = 640
BR = 1000
GRID = N // BR

_MESH = plsc.VectorSubcoreMesh(core_axis_name="c", subcore_axis_name="s")
_SC_PARAMS = pltpu.CompilerParams(needs_layout_passes=False)


def _wid():
    return lax.axis_index("s") * NC + lax.axis_index("c")


@functools.partial(
    pl.kernel,
    out_type=jax.ShapeDtypeStruct((NC, N, HID), jnp.float32),
    mesh=_MESH,
    compiler_params=_SC_PARAMS,
    scratch_types=[
        pltpu.VMEM_SHARED((N, HID), jnp.float32),
        pltpu.VMEM((CH,), jnp.int32),
        pltpu.VMEM((CH, HID), jnp.float32),
    ],
)
def _sc_degree(dst_hbm, z128_hbm, ones_hbm, out_hbm, acc, didx, ones_v):
    cid = lax.axis_index("c")
    sid = lax.axis_index("s")
    wid = _wid()
    rows = pl.ds(sid * SLAB_STEP, SLAB)
    pltpu.sync_copy(z128_hbm.at[rows], acc.at[rows])
    pltpu.sync_copy(ones_hbm, ones_v)
    plsc.subcore_barrier()

    def body(j, c):
        pltpu.sync_copy(dst_hbm.at[wid, j], didx)
        pltpu.sync_copy(ones_v, acc.at[didx], add=True)
        return c

    lax.fori_loop(0, NCH, body, 0)
    plsc.subcore_barrier()
    pltpu.sync_copy(acc.at[rows], out_hbm.at[cid, rows])


@functools.partial(
    pl.kernel,
    out_type=jax.ShapeDtypeStruct((NC, N, HID), jnp.float32),
    mesh=_MESH,
    compiler_params=_SC_PARAMS,
    scratch_types=[
        pltpu.VMEM_SHARED((N, HID), jnp.float32),
        pltpu.VMEM((NCH, CH), jnp.int32),
        pltpu.VMEM((CH,), jnp.int32),
        pltpu.VMEM((CH, HID), jnp.float32),
        pltpu.SemaphoreType.DMA,
    ],
)
def _sc_edge_agg(y_hbm, src_hbm, dst_hbm, z128_hbm, out_hbm,
                 acc, sidx, didx, buf, sem):
    cid = lax.axis_index("c")
    sid = lax.axis_index("s")
    wid = _wid()
    rows = pl.ds(sid * SLAB_STEP, SLAB)
    pltpu.sync_copy(z128_hbm.at[rows], acc.at[rows])
    pltpu.sync_copy(src_hbm.at[wid], sidx)
    plsc.subcore_barrier()

    def body(j, c):
        pltpu.async_copy(y_hbm.at[sidx.at[j]], buf, sem).wait()
        pltpu.sync_copy(dst_hbm.at[wid, j], didx)
        pltpu.sync_copy(buf, acc.at[didx], add=True)
        return c

    lax.fori_loop(0, NCH, body, 0)
    plsc.subcore_barrier()
    pltpu.sync_copy(acc.at[rows], out_hbm.at[cid, rows])


@functools.partial(
    pl.kernel,
    out_type=(
        jax.ShapeDtypeStruct((NW, NCH, NHEAD, CH), jnp.float32),
        jax.ShapeDtypeStruct((NC, N, HID), jnp.float32),
    ),
    mesh=_MESH,
    compiler_params=_SC_PARAMS,
    scratch_types=[
        pltpu.VMEM_SHARED((N, HID), jnp.float32),
        pltpu.VMEM((CH,), jnp.int32),
        pltpu.VMEM((CH,), jnp.int32),
        pltpu.VMEM((CH, HID), jnp.float32),
        pltpu.VMEM((CH, HID), jnp.float32),
        pltpu.VMEM((CH, HID), jnp.float32),
        pltpu.VMEM((NHEAD, CH), jnp.float32),
        pltpu.SemaphoreType.DMA,
        pltpu.SemaphoreType.DMA,
    ],
)
def _sc_gat_logits(als_hbm, ald_hbm, src_hbm, dst_hbm, z128_hbm,
                   ex_hbm, den_hbm,
                   acc, sidx, didx, abuf, bbuf, exb, exT, sema, semb):
    cid = lax.axis_index("c")
    sid = lax.axis_index("s")
    wid = _wid()
    rows = pl.ds(sid * SLAB_STEP, SLAB)
    pltpu.sync_copy(z128_hbm.at[rows], acc.at[rows])
    plsc.subcore_barrier()
    iota16 = lax.iota(jnp.int32, 16)
    lane_mask = iota16 < NHEAD

    def body(j, c):
        pltpu.sync_copy(src_hbm.at[wid, j], sidx)
        pltpu.sync_copy(dst_hbm.at[wid, j], didx)
        ca = pltpu.async_copy(als_hbm.at[sidx], abuf, sema)
        cb = pltpu.async_copy(ald_hbm.at[didx], bbuf, semb)
        ca.wait()
        cb.wait()

        def edge(r, c2):
            v = abuf[r, pl.ds(0, 16)] + bbuf[r, pl.ds(0, 16)]
            ex = jnp.exp(jnp.maximum(v, 0.2 * v))
            exb[r, pl.ds(0, 16)] = ex
            plsc.store_scatter(exT, [iota16, jnp.full((16,), r, jnp.int32)],
                               ex, mask=lane_mask)
            return c2

        lax.fori_loop(0, CH, edge, 0)
        pltpu.sync_copy(exT, ex_hbm.at[wid, j])
        pltpu.sync_copy(exb, acc.at[didx], add=True)
        return c

    lax.fori_loop(0, NCH, body, 0)
    plsc.subcore_barrier()
    pltpu.sync_copy(acc.at[rows], den_hbm.at[cid, rows])


def _make_gat_head_agg(head):
    @functools.partial(
        pl.kernel,
        out_type=jax.ShapeDtypeStruct((NC, N, HID), jnp.float32),
        mesh=_MESH,
        compiler_params=_SC_PARAMS,
        scratch_types=[
            pltpu.VMEM_SHARED((N, HID), jnp.float32),
            pltpu.VMEM((CH,), jnp.int32),
            pltpu.VMEM((CH,), jnp.int32),
            pltpu.VMEM((CH,), jnp.float32),
            pltpu.VMEM((CH, HID), jnp.float32),
            pltpu.SemaphoreType.DMA,
        ],
    )
    def _sc_gat_head(g4_hbm, ex_hbm, src_hbm, dst_hbm, z128_hbm, out_hbm,
                     acc, sidx4, didx, exc, gbuf, sem):
        cid = lax.axis_index("c")
        sid = lax.axis_index("s")
        wid = _wid()
        rows = pl.ds(sid * SLAB_STEP, SLAB)
        pltpu.sync_copy(z128_hbm.at[rows], acc.at[rows])
        plsc.subcore_barrier()

        def body(j, c):
            pltpu.sync_copy(src_hbm.at[wid, j], sidx4)
            pltpu.sync_copy(dst_hbm.at[wid, j], didx)
            pltpu.sync_copy(ex_hbm.at[wid, j, head], exc)

            def grp(k, c2):
                sl = pl.ds(k * 16, 16)
                sidx4[sl] = sidx4[sl] * NHEAD + head
                return c2

            lax.fori_loop(0, NGRP, grp, 0)
            pltpu.async_copy(g4_hbm.at[sidx4], gbuf, sem).wait()

            def edge(r, c2):
                w = plsc.load_gather(exc, [jnp.full((16,), r, jnp.int32)])
                for v in range(HID // 16):
                    sl = pl.ds(v * 16, 16)
                    gbuf[r, sl] = gbuf[r, sl] * w
                return c2

            lax.fori_loop(0, CH, edge, 0)
            pltpu.sync_copy(gbuf, acc.at[didx], add=True)
            return c

        lax.fori_loop(0, NCH, body, 0)
        plsc.subcore_barrier()
        pltpu.sync_copy(acc.at[rows], out_hbm.at[cid, rows])

    return _sc_gat_head


_sc_gat_heads = [_make_gat_head_agg(h) for h in range(NHEAD)]


def _tc_first(deg_part, x, W):
    def body(p_ref, x_ref, w_ref, y_ref, d_ref):
        deg = 1.0 + p_ref[0, :, 0:1] + p_ref[1, :, 0:1]
        dinv = lax.rsqrt(deg)
        y_ref[...] = dinv * jnp.dot(x_ref[...], w_ref[...],
                                    preferred_element_type=jnp.float32)
        d_ref[...] = dinv

    return pl.pallas_call(
        body,
        grid=(GRID,),
        in_specs=[
            pl.BlockSpec((NC, BR, HID), lambda i: (0, i, 0)),
            pl.BlockSpec((BR, HID), lambda i: (i, 0)),
            pl.BlockSpec((HID, HID), lambda i: (0, 0)),
        ],
        out_specs=[
            pl.BlockSpec((BR, HID), lambda i: (i, 0)),
            pl.BlockSpec((BR, 1), lambda i: (i, 0)),
        ],
        out_shape=[
            jax.ShapeDtypeStruct((N, HID), jnp.float32),
            jax.ShapeDtypeStruct((N, 1), jnp.float32),
        ],
    )(deg_part, x, W)


def _tc_combine_mm(part, Y, dinv, b, W):
    def body(p_ref, y_ref, d_ref, b_ref, w_ref, o_ref):
        dinv = d_ref[...]
        h = jnp.maximum(dinv * (p_ref[0] + p_ref[1] + y_ref[...]) + b_ref[...],
                        0.0)
        o_ref[...] = dinv * jnp.dot(h, w_ref[...],
                                    preferred_element_type=jnp.float32)

    return pl.pallas_call(
        body,
        grid=(GRID,),
        in_specs=[
            pl.BlockSpec((NC, BR, HID), lambda i: (0, i, 0)),
            pl.BlockSpec((BR, HID), lambda i: (i, 0)),
            pl.BlockSpec((BR, 1), lambda i: (i, 0)),
            pl.BlockSpec((1, HID), lambda i: (0, 0)),
            pl.BlockSpec((HID, HID), lambda i: (0, 0)),
        ],
        out_specs=pl.BlockSpec((BR, HID), lambda i: (i, 0)),
        out_shape=jax.ShapeDtypeStruct((N, HID), jnp.float32),
    )(part, Y, dinv, b, W)


def _tc_gat_proj(part, Y, dinv, b, Wg, asr, adr):
    def body(p_ref, y_ref, d_ref, b_ref, w_ref, as_ref, ad_ref,
             g_ref, als_ref, ald_ref, exl_ref):
        dinv = d_ref[...]
        h = jnp.maximum(dinv * (p_ref[0] + p_ref[1] + y_ref[...]) + b_ref[...],
                        0.0)
        g = jnp.dot(h, w_ref[...], preferred_element_type=jnp.float32)
        g_ref[...] = g
        ts = g * as_ref[...]
        td = g * ad_ref[...]
        z = jnp.zeros((BR, HID - NHEAD), jnp.float32)
        als = jnp.concatenate(
            [jnp.sum(ts[:, h0 * HID:(h0 + 1) * HID], axis=1, keepdims=True)
             for h0 in range(NHEAD)] + [z], axis=1)
        ald = jnp.concatenate(
            [jnp.sum(td[:, h0 * HID:(h0 + 1) * HID], axis=1, keepdims=True)
             for h0 in range(NHEAD)] + [z], axis=1)
        als_ref[...] = als
        ald_ref[...] = ald
        v = als[:, 0:NHEAD] + ald[:, 0:NHEAD]
        exl_ref[...] = jnp.exp(jnp.maximum(v, 0.2 * v))

    return pl.pallas_call(
        body,
        grid=(GRID,),
        in_specs=[
            pl.BlockSpec((NC, BR, HID), lambda i: (0, i, 0)),
            pl.BlockSpec((BR, HID), lambda i: (i, 0)),
            pl.BlockSpec((BR, 1), lambda i: (i, 0)),
            pl.BlockSpec((1, HID), lambda i: (0, 0)),
            pl.BlockSpec((HID, GDIM), lambda i: (0, 0)),
            pl.BlockSpec((1, GDIM), lambda i: (0, 0)),
            pl.BlockSpec((1, GDIM), lambda i: (0, 0)),
        ],
        out_specs=[
            pl.BlockSpec((BR, GDIM), lambda i: (i, 0)),
            pl.BlockSpec((BR, HID), lambda i: (i, 0)),
            pl.BlockSpec((BR, HID), lambda i: (i, 0)),
            pl.BlockSpec((BR, NHEAD), lambda i: (i, 0)),
        ],
        out_shape=[
            jax.ShapeDtypeStruct((N, GDIM), jnp.float32),
            jax.ShapeDtypeStruct((N, HID), jnp.float32),
            jax.ShapeDtypeStruct((N, HID), jnp.float32),
            jax.ShapeDtypeStruct((N, NHEAD), jnp.float32),
        ],
    )(part, Y, dinv, b, Wg, asr, adr)


def _tc_rden(den_part, exl, g):
    def body(p_ref, e_ref, g_ref, rw_ref, ms_ref):
        den = p_ref[0, :, 0:NHEAD] + p_ref[1, :, 0:NHEAD] + e_ref[...]
        rw = 0.25 / jnp.maximum(den, 1e-30)
        rw_ref[...] = rw
        ws = e_ref[...] * rw
        g = g_ref[...]
        msg = ws[:, 0:1] * g[:, 0:HID]
        for h in range(1, NHEAD):
            msg = msg + ws[:, h:h + 1] * g[:, h * HID:(h + 1) * HID]
        ms_ref[...] = msg

    return pl.pallas_call(
        body,
        grid=(GRID,),
        in_specs=[
            pl.BlockSpec((NC, BR, HID), lambda i: (0, i, 0)),
            pl.BlockSpec((BR, NHEAD), lambda i: (i, 0)),
            pl.BlockSpec((BR, GDIM), lambda i: (i, 0)),
        ],
        out_specs=[
            pl.BlockSpec((BR, NHEAD), lambda i: (i, 0)),
            pl.BlockSpec((BR, HID), lambda i: (i, 0)),
        ],
        out_shape=[
            jax.ShapeDtypeStruct((N, NHEAD), jnp.float32),
            jax.ShapeDtypeStruct((N, HID), jnp.float32),
        ],
    )(den_part, exl, g)


def _tc_pool(b0, b1, b2, b3, rw, msg_self, b, batch2):
    def body(b0_ref, b1_ref, b2_ref, b3_ref, rw_ref, m_ref, b_ref, bt_ref,
             s_ref, x_ref, c_ref):
        @pl.when(pl.program_id(0) == 0)
        def _():
            s_ref[...] = jnp.zeros_like(s_ref)
            x_ref[...] = jnp.zeros_like(x_ref)
            c_ref[...] = jnp.zeros_like(c_ref)

        rw = rw_ref[...]
        agg = m_ref[...]
        for h, br in enumerate([b0_ref, b1_ref, b2_ref, b3_ref]):
            agg = agg + rw[:, h:h + 1] * (br[0] + br[1])
        h4 = jnp.maximum(agg + b_ref[...], 0.0)
        ids = lax.broadcasted_iota(jnp.int32, (BR, NG), 1)
        mf = (bt_ref[...] == ids).astype(jnp.float32)
        s_ref[...] += lax.dot_general(mf, h4, (((0,), (0,)), ((), ())),
                                      preferred_element_type=jnp.float32)
        onec = jnp.ones((BR, 1), jnp.float32)
        c_ref[...] += lax.dot_general(mf, onec, (((0,), (0,)), ((), ())),
                                      preferred_element_type=jnp.float32)
        rows = [jnp.max(mf[:, g0:g0 + 1] * h4, axis=0, keepdims=True)
                for g0 in range(NG)]
        cur = jnp.concatenate(rows, axis=0)
        x_ref[...] = jnp.maximum(x_ref[...], cur)

    part_spec = pl.BlockSpec((NC, BR, HID), lambda i: (0, i, 0))
    return pl.pallas_call(
        body,
        grid=(GRID,),
        in_specs=[
            part_spec, part_spec, part_spec, part_spec,
            pl.BlockSpec((BR, NHEAD), lambda i: (i, 0)),
            pl.BlockSpec((BR, HID), lambda i: (i, 0)),
            pl.BlockSpec((1, HID), lambda i: (0, 0)),
            pl.BlockSpec((BR, 1), lambda i: (i, 0)),
        ],
        out_specs=[
            pl.BlockSpec((NG, HID), lambda i: (0, 0)),
            pl.BlockSpec((NG, HID), lambda i: (0, 0)),
            pl.BlockSpec((NG, 1), lambda i: (0, 0)),
        ],
        out_shape=[
            jax.ShapeDtypeStruct((NG, HID), jnp.float32),
            jax.ShapeDtypeStruct((NG, HID), jnp.float32),
            jax.ShapeDtypeStruct((NG, 1), jnp.float32),
        ],
    )(b0, b1, b2, b3, rw, msg_self, b, batch2)


def _tc_classifier(sp, xp, cnt, W1, b1, W2, b2, W3, b3):
    def body(s_ref, x_ref, c_ref, w1_ref, b1_ref, w2_ref, b2_ref,
             w3_ref, b3_ref, o_ref):
        cnt = c_ref[...]
        mean = s_ref[...] / jnp.maximum(cnt, 1.0)
        mx = jnp.where(cnt > 0.0, x_ref[...], 0.0)
        rep = jnp.concatenate([mean, mx], axis=1)
        z = jnp.maximum(jnp.dot(rep, w1_ref[...],
                                preferred_element_type=jnp.float32)
                        + b1_ref[...], 0.0)
        z = jnp.maximum(jnp.dot(z, w2_ref[...],
                                preferred_element_type=jnp.float32)
                        + b2_ref[...], 0.0)
        lg = jnp.dot(z, w3_ref[...], preferred_element_type=jnp.float32) \
            + b3_ref[...]
        m = jnp.max(lg, axis=1, keepdims=True)
        e = jnp.exp(lg - m)
        o_ref[...] = e / jnp.sum(e, axis=1, keepdims=True)

    return pl.pallas_call(
        body,
        out_shape=jax.ShapeDtypeStruct((NG, 3), jnp.float32),
    )(sp, xp, cnt, W1, b1, W2, b2, W3, b3)


def kernel(x, edge_index, batch, W_gcn1, b_gcn1, W_gcn2, b_gcn2, W_gcn3,
           b_gcn3, W_gat, a_src, a_dst, b_gat, W_c1, b_c1, W_c2, b_c2,
           W_c3, b_c3):
    src3 = edge_index[0].reshape(NW, NCH, CH)
    dst3 = edge_index[1].reshape(NW, NCH, CH)
    z128 = jnp.zeros((N, HID), jnp.float32)
    ones128 = jnp.ones((CH, HID), jnp.float32)
    batch2 = batch.reshape(N, 1)

    deg_part = _sc_degree(dst3, z128, ones128)
    Y1, dinv = _tc_first(deg_part, x, W_gcn1)
    P1 = _sc_edge_agg(Y1, src3, dst3, z128)
    Y2 = _tc_combine_mm(P1, Y1, dinv, b_gcn1.reshape(1, HID), W_gcn2)
    P2 = _sc_edge_agg(Y2, src3, dst3, z128)
    Y3 = _tc_combine_mm(P2, Y2, dinv, b_gcn2.reshape(1, HID), W_gcn3)
    P3 = _sc_edge_agg(Y3, src3, dst3, z128)
    g, als, ald, exl = _tc_gat_proj(P3, Y3, dinv, b_gcn3.reshape(1, HID),
                                    W_gat, a_src.reshape(1, GDIM),
                                    a_dst.reshape(1, GDIM))
    ex_e, den_part = _sc_gat_logits(als, ald, src3, dst3, z128)
    rw, msg_self = _tc_rden(den_part, exl, g)
    g4 = g.reshape(N * NHEAD, HID)
    bs = [_sc_gat_heads[h](g4, ex_e, src3, dst3, z128) for h in range(NHEAD)]
    sp, xp, cnt = _tc_pool(bs[0], bs[1], bs[2], bs[3], rw, msg_self,
                           b_gat.reshape(1, HID), batch2)
    return _tc_classifier(sp, xp, cnt, W_c1, b_c1.reshape(1, HID),
                          W_c2, b_c2.reshape(1, HID // 2),
                          W_c3, b_c3.reshape(1, 3))


# speedup vs baseline: 21.2323x; 1.3936x over previous
import functools
import jax
import jax.numpy as jnp
from jax import lax
from jax.experimental import pallas as pl
from jax.experimental.pallas import tpu as pltpu
from jax.experimental.pallas import tpu_sc as plsc

N = 10000
E = 320000
HID = 128
NHEAD = 4
GDIM = NHEAD * HID
NG = 64
NC = 2
NS = 16
NW = NC * NS
EW = E // NW
CH = 80
NCH = EW // CH
NGRP = CH // 16
SLAB_STEP = 624
SLAB = 640
BR = 1000
GRID = N // BR

_MESH = plsc.VectorSubcoreMesh(core_axis_name="c", subcore_axis_name="s")
_SC_PARAMS = pltpu.CompilerParams(needs_layout_passes=False)


def _wid():
    return lax.axis_index("s") * NC + lax.axis_index("c")


@functools.partial(
    pl.kernel,
    out_type=jax.ShapeDtypeStruct((NC, N, HID), jnp.float32),
    mesh=_MESH,
    compiler_params=_SC_PARAMS,
    scratch_types=[
        pltpu.VMEM_SHARED((N, HID), jnp.float32),
        pltpu.VMEM((CH,), jnp.int32),
        pltpu.VMEM((CH,), jnp.int32),
        pltpu.VMEM((CH, HID), jnp.float32),
        pltpu.SemaphoreType.DMA,
        pltpu.SemaphoreType.DMA,
    ],
)
def _sc_degree(dst_hbm, z128_hbm, ones_hbm, out_hbm, acc, d0, d1, ones_v,
               sem0, sem1):
    cid = lax.axis_index("c")
    sid = lax.axis_index("s")
    wid = _wid()
    ebase = wid * EW
    rows = pl.ds(sid * SLAB_STEP, SLAB)
    pltpu.sync_copy(z128_hbm.at[rows], acc.at[rows])
    pltpu.sync_copy(ones_hbm, ones_v)
    plsc.subcore_barrier()

    def pair(j2, c):
        a = 2 * j2
        pltpu.sync_copy(dst_hbm.at[pl.ds(ebase + a * CH, CH)], d0)
        pltpu.async_copy(ones_v, acc.at[d0], sem0, add=True)
        pltpu.sync_copy(dst_hbm.at[pl.ds(ebase + (a + 1) * CH, CH)], d1)
        pltpu.async_copy(ones_v, acc.at[d1], sem1, add=True)
        pltpu.make_async_copy(ones_v, acc.at[d0], sem0).wait()
        pltpu.make_async_copy(ones_v, acc.at[d1], sem1).wait()
        return c

    lax.fori_loop(0, NCH // 2, pair, 0)
    pltpu.sync_copy(dst_hbm.at[pl.ds(ebase + (NCH - 1) * CH, CH)], d0)
    pltpu.sync_copy(ones_v, acc.at[d0], add=True)
    plsc.subcore_barrier()
    pltpu.sync_copy(acc.at[rows], out_hbm.at[cid, rows])


@functools.partial(
    pl.kernel,
    out_type=jax.ShapeDtypeStruct((NC, N, HID), jnp.float32),
    mesh=_MESH,
    compiler_params=_SC_PARAMS,
    scratch_types=[
        pltpu.VMEM_SHARED((N, HID), jnp.float32),
        pltpu.VMEM((EW,), jnp.int32),
        pltpu.VMEM((CH,), jnp.int32),
        pltpu.VMEM((CH, HID), jnp.float32),
        pltpu.VMEM((CH, HID), jnp.float32),
        pltpu.SemaphoreType.DMA,
        pltpu.SemaphoreType.DMA,
    ],
)
def _sc_edge_agg(y_hbm, src_hbm, dst_hbm, z128_hbm, out_hbm,
                 acc, sidx, didxc, buf0, buf1, sem0, sem1):
    cid = lax.axis_index("c")
    sid = lax.axis_index("s")
    wid = _wid()
    ebase = wid * EW
    rows = pl.ds(sid * SLAB_STEP, SLAB)
    pltpu.sync_copy(z128_hbm.at[rows], acc.at[rows])
    pltpu.sync_copy(src_hbm.at[pl.ds(ebase, EW)], sidx)
    plsc.subcore_barrier()

    def sl(n):
        return pl.ds(n * CH, CH)

    pltpu.async_copy(y_hbm.at[sidx.at[sl(0)]], buf0, sem0)

    def pair(j2, c):
        a = 2 * j2
        pltpu.async_copy(y_hbm.at[sidx.at[sl(a + 1)]], buf1, sem1)
        pltpu.sync_copy(dst_hbm.at[pl.ds(ebase + a * CH, CH)], didxc)
        pltpu.make_async_copy(y_hbm.at[sidx.at[sl(a)]], buf0, sem0).wait()
        pltpu.sync_copy(buf0, acc.at[didxc], add=True)
        pltpu.async_copy(y_hbm.at[sidx.at[sl(a + 2)]], buf0, sem0)
        pltpu.sync_copy(dst_hbm.at[pl.ds(ebase + (a + 1) * CH, CH)], didxc)
        pltpu.make_async_copy(y_hbm.at[sidx.at[sl(a + 1)]], buf1, sem1).wait()
        pltpu.sync_copy(buf1, acc.at[didxc], add=True)
        return c

    lax.fori_loop(0, NCH // 2, pair, 0)
    pltpu.sync_copy(dst_hbm.at[pl.ds(ebase + (NCH - 1) * CH, CH)], didxc)
    pltpu.make_async_copy(y_hbm.at[sidx.at[sl(NCH - 1)]], buf0, sem0).wait()
    pltpu.sync_copy(buf0, acc.at[didxc], add=True)
    plsc.subcore_barrier()
    pltpu.sync_copy(acc.at[rows], out_hbm.at[cid, rows])


@functools.partial(
    pl.kernel,
    out_type=(
        jax.ShapeDtypeStruct((NW, NCH, NHEAD, CH), jnp.float32),
        jax.ShapeDtypeStruct((NC, N, HID), jnp.float32),
    ),
    mesh=_MESH,
    compiler_params=_SC_PARAMS,
    scratch_types=[
        pltpu.VMEM_SHARED((N, HID), jnp.float32),
        pltpu.VMEM((CH,), jnp.int32),
        pltpu.VMEM((CH,), jnp.int32),
        pltpu.VMEM((CH, HID), jnp.float32),
        pltpu.VMEM((CH, HID), jnp.float32),
        pltpu.VMEM((CH, HID), jnp.float32),
        pltpu.VMEM((NHEAD, CH), jnp.float32),
        pltpu.SemaphoreType.DMA,
        pltpu.SemaphoreType.DMA,
    ],
)
def _sc_gat_logits(als_hbm, ald_hbm, src_hbm, dst_hbm, z128_hbm,
                   ex_hbm, den_hbm,
                   acc, sidx, didx, abuf, bbuf, exb, exT, sema, semb):
    cid = lax.axis_index("c")
    sid = lax.axis_index("s")
    wid = _wid()
    ebase = wid * EW
    rows = pl.ds(sid * SLAB_STEP, SLAB)
    pltpu.sync_copy(z128_hbm.at[rows], acc.at[rows])
    plsc.subcore_barrier()
    iota16 = lax.iota(jnp.int32, 16)
    lane_mask = iota16 < NHEAD

    def body(j, c):
        pltpu.sync_copy(src_hbm.at[pl.ds(ebase + j * CH, CH)], sidx)
        pltpu.sync_copy(dst_hbm.at[pl.ds(ebase + j * CH, CH)], didx)
        ca = pltpu.async_copy(als_hbm.at[sidx], abuf, sema)
        cb = pltpu.async_copy(ald_hbm.at[didx], bbuf, semb)
        ca.wait()
        cb.wait()

        def edge(r, c2):
            v = abuf[r, pl.ds(0, 16)] + bbuf[r, pl.ds(0, 16)]
            ex = jnp.exp(jnp.maximum(v, 0.2 * v))
            exb[r, pl.ds(0, 16)] = ex
            plsc.store_scatter(exT, [iota16, jnp.full((16,), r, jnp.int32)],
                               ex, mask=lane_mask)
            return c2

        lax.fori_loop(0, CH, edge, 0)
        pltpu.sync_copy(exT, ex_hbm.at[wid, j])
        pltpu.sync_copy(exb, acc.at[didx], add=True)
        return c

    lax.fori_loop(0, NCH, body, 0)
    plsc.subcore_barrier()
    pltpu.sync_copy(acc.at[rows], den_hbm.at[cid, rows])


def _make_gat_head_agg(head):
    @functools.partial(
        pl.kernel,
        out_type=jax.ShapeDtypeStruct((NC, N, HID), jnp.float32),
        mesh=_MESH,
        compiler_params=_SC_PARAMS,
        scratch_types=[
            pltpu.VMEM_SHARED((N, HID), jnp.float32),
            pltpu.VMEM((CH,), jnp.int32),
            pltpu.VMEM((CH,), jnp.int32),
            pltpu.VMEM((CH,), jnp.int32),
            pltpu.VMEM((1, CH), jnp.float32),
            pltpu.VMEM((1, CH), jnp.float32),
            pltpu.VMEM((CH, HID), jnp.float32),
            pltpu.VMEM((CH, HID), jnp.float32),
            pltpu.SemaphoreType.DMA,
            pltpu.SemaphoreType.DMA,
        ],
    )
    def _sc_gat_head(g4_hbm, ex_hbm, src_hbm, dst_hbm, z128_hbm, out_hbm,
                     acc, didxc, si40, si41, exc0, exc1, buf0, buf1,
                     sem0, sem1):
        cid = lax.axis_index("c")
        sid = lax.axis_index("s")
        wid = _wid()
        ebase = wid * EW
        rows = pl.ds(sid * SLAB_STEP, SLAB)
        pltpu.sync_copy(z128_hbm.at[rows], acc.at[rows])
        plsc.subcore_barrier()

        def mkidx(n, si4):
            pltpu.sync_copy(src_hbm.at[pl.ds(ebase + n * CH, CH)], si4)

            def grp(k, c2):
                sl = pl.ds(k * 16, 16)
                si4[sl] = si4[sl] * NHEAD + head
                return c2
            lax.fori_loop(0, NGRP, grp, 0)

        def scale(buf, exc):
            z16i = jnp.zeros((16,), jnp.int32)

            def edge(r, c2):
                w = plsc.load_gather(exc, [z16i, jnp.full((16,), r, jnp.int32)])
                for v in range(HID // 16):
                    sl = pl.ds(v * 16, 16)
                    buf[r, sl] = buf[r, sl] * w
                return c2
            lax.fori_loop(0, CH, edge, 0)

        mkidx(0, si40)
        pltpu.async_copy(g4_hbm.at[si40], buf0, sem0)
        pltpu.sync_copy(ex_hbm.at[wid, 0, pl.ds(head, 1)], exc0)

        def pair(j2, c):
            a = 2 * j2
            mkidx(a + 1, si41)
            pltpu.async_copy(g4_hbm.at[si41], buf1, sem1)
            pltpu.sync_copy(ex_hbm.at[wid, a + 1, pl.ds(head, 1)], exc1)
            pltpu.sync_copy(dst_hbm.at[pl.ds(ebase + a * CH, CH)], didxc)
            pltpu.make_async_copy(g4_hbm.at[si40], buf0, sem0).wait()
            scale(buf0, exc0)
            pltpu.sync_copy(buf0, acc.at[didxc], add=True)
            mkidx(a + 2, si40)
            pltpu.async_copy(g4_hbm.at[si40], buf0, sem0)
            pltpu.sync_copy(ex_hbm.at[wid, a + 2, pl.ds(head, 1)], exc0)
            pltpu.sync_copy(dst_hbm.at[pl.ds(ebase + (a + 1) * CH, CH)], didxc)
            pltpu.make_async_copy(g4_hbm.at[si41], buf1, sem1).wait()
            scale(buf1, exc1)
            pltpu.sync_copy(buf1, acc.at[didxc], add=True)
            return c

        lax.fori_loop(0, NCH // 2, pair, 0)
        pltpu.sync_copy(dst_hbm.at[pl.ds(ebase + (NCH - 1) * CH, CH)], didxc)
        pltpu.make_async_copy(g4_hbm.at[si40], buf0, sem0).wait()
        scale(buf0, exc0)
        pltpu.sync_copy(buf0, acc.at[didxc], add=True)
        plsc.subcore_barrier()
        pltpu.sync_copy(acc.at[rows], out_hbm.at[cid, rows])

    return _sc_gat_head


_sc_gat_heads = [_make_gat_head_agg(h) for h in range(NHEAD)]


def _tc_first(deg_part, x, W):
    def body(p_ref, x_ref, w_ref, y_ref, d_ref):
        deg = 1.0 + p_ref[0, :, 0:1] + p_ref[1, :, 0:1]
        dinv = lax.rsqrt(deg)
        y_ref[...] = dinv * jnp.dot(x_ref[...], w_ref[...],
                                    preferred_element_type=jnp.float32)
        d_ref[...] = dinv

    return pl.pallas_call(
        body,
        grid=(GRID,),
        in_specs=[
            pl.BlockSpec((NC, BR, HID), lambda i: (0, i, 0)),
            pl.BlockSpec((BR, HID), lambda i: (i, 0)),
            pl.BlockSpec((HID, HID), lambda i: (0, 0)),
        ],
        out_specs=[
            pl.BlockSpec((BR, HID), lambda i: (i, 0)),
            pl.BlockSpec((BR, 1), lambda i: (i, 0)),
        ],
        out_shape=[
            jax.ShapeDtypeStruct((N, HID), jnp.float32),
            jax.ShapeDtypeStruct((N, 1), jnp.float32),
        ],
    )(deg_part, x, W)


def _tc_combine_mm(part, Y, dinv, b, W):
    def body(p_ref, y_ref, d_ref, b_ref, w_ref, o_ref):
        dinv = d_ref[...]
        h = jnp.maximum(dinv * (p_ref[0] + p_ref[1] + y_ref[...]) + b_ref[...],
                        0.0)
        o_ref[...] = dinv * jnp.dot(h, w_ref[...],
                                    preferred_element_type=jnp.float32)

    return pl.pallas_call(
        body,
        grid=(GRID,),
        in_specs=[
            pl.BlockSpec((NC, BR, HID), lambda i: (0, i, 0)),
            pl.BlockSpec((BR, HID), lambda i: (i, 0)),
            pl.BlockSpec((BR, 1), lambda i: (i, 0)),
            pl.BlockSpec((1, HID), lambda i: (0, 0)),
            pl.BlockSpec((HID, HID), lambda i: (0, 0)),
        ],
        out_specs=pl.BlockSpec((BR, HID), lambda i: (i, 0)),
        out_shape=jax.ShapeDtypeStruct((N, HID), jnp.float32),
    )(part, Y, dinv, b, W)


def _tc_gat_proj(part, Y, dinv, b, Wg, asr, adr):
    def body(p_ref, y_ref, d_ref, b_ref, w_ref, as_ref, ad_ref,
             g_ref, als_ref, ald_ref, exl_ref):
        dinv = d_ref[...]
        h = jnp.maximum(dinv * (p_ref[0] + p_ref[1] + y_ref[...]) + b_ref[...],
                        0.0)
        g = jnp.dot(h, w_ref[...], preferred_element_type=jnp.float32)
        g_ref[...] = g
        ts = g * as_ref[...]
        td = g * ad_ref[...]
        z = jnp.zeros((BR, HID - NHEAD), jnp.float32)
        als = jnp.concatenate(
            [jnp.sum(ts[:, h0 * HID:(h0 + 1) * HID], axis=1, keepdims=True)
             for h0 in range(NHEAD)] + [z], axis=1)
        ald = jnp.concatenate(
            [jnp.sum(td[:, h0 * HID:(h0 + 1) * HID], axis=1, keepdims=True)
             for h0 in range(NHEAD)] + [z], axis=1)
        als_ref[...] = als
        ald_ref[...] = ald
        v = als[:, 0:NHEAD] + ald[:, 0:NHEAD]
        exl_ref[...] = jnp.exp(jnp.maximum(v, 0.2 * v))

    return pl.pallas_call(
        body,
        grid=(GRID,),
        in_specs=[
            pl.BlockSpec((NC, BR, HID), lambda i: (0, i, 0)),
            pl.BlockSpec((BR, HID), lambda i: (i, 0)),
            pl.BlockSpec((BR, 1), lambda i: (i, 0)),
            pl.BlockSpec((1, HID), lambda i: (0, 0)),
            pl.BlockSpec((HID, GDIM), lambda i: (0, 0)),
            pl.BlockSpec((1, GDIM), lambda i: (0, 0)),
            pl.BlockSpec((1, GDIM), lambda i: (0, 0)),
        ],
        out_specs=[
            pl.BlockSpec((BR, GDIM), lambda i: (i, 0)),
            pl.BlockSpec((BR, HID), lambda i: (i, 0)),
            pl.BlockSpec((BR, HID), lambda i: (i, 0)),
            pl.BlockSpec((BR, NHEAD), lambda i: (i, 0)),
        ],
        out_shape=[
            jax.ShapeDtypeStruct((N, GDIM), jnp.float32),
            jax.ShapeDtypeStruct((N, HID), jnp.float32),
            jax.ShapeDtypeStruct((N, HID), jnp.float32),
            jax.ShapeDtypeStruct((N, NHEAD), jnp.float32),
        ],
    )(part, Y, dinv, b, Wg, asr, adr)


def _tc_rden(den_part, exl, g):
    def body(p_ref, e_ref, g_ref, rw_ref, ms_ref):
        den = p_ref[0, :, 0:NHEAD] + p_ref[1, :, 0:NHEAD] + e_ref[...]
        rw = 0.25 / jnp.maximum(den, 1e-30)
        rw_ref[...] = rw
        ws = e_ref[...] * rw
        g = g_ref[...]
        msg = ws[:, 0:1] * g[:, 0:HID]
        for h in range(1, NHEAD):
            msg = msg + ws[:, h:h + 1] * g[:, h * HID:(h + 1) * HID]
        ms_ref[...] = msg

    return pl.pallas_call(
        body,
        grid=(GRID,),
        in_specs=[
            pl.BlockSpec((NC, BR, HID), lambda i: (0, i, 0)),
            pl.BlockSpec((BR, NHEAD), lambda i: (i, 0)),
            pl.BlockSpec((BR, GDIM), lambda i: (i, 0)),
        ],
        out_specs=[
            pl.BlockSpec((BR, NHEAD), lambda i: (i, 0)),
            pl.BlockSpec((BR, HID), lambda i: (i, 0)),
        ],
        out_shape=[
            jax.ShapeDtypeStruct((N, NHEAD), jnp.float32),
            jax.ShapeDtypeStruct((N, HID), jnp.float32),
        ],
    )(den_part, exl, g)


def _tc_pool(b0, b1, b2, b3, rw, msg_self, b, batch2):
    def body(b0_ref, b1_ref, b2_ref, b3_ref, rw_ref, m_ref, b_ref, bt_ref,
             s_ref, x_ref, c_ref):
        @pl.when(pl.program_id(0) == 0)
        def _():
            s_ref[...] = jnp.zeros_like(s_ref)
            x_ref[...] = jnp.zeros_like(x_ref)
            c_ref[...] = jnp.zeros_like(c_ref)

        rw = rw_ref[...]
        agg = m_ref[...]
        for h, br in enumerate([b0_ref, b1_ref, b2_ref, b3_ref]):
            agg = agg + rw[:, h:h + 1] * (br[0] + br[1])
        h4 = jnp.maximum(agg + b_ref[...], 0.0)
        ids = lax.broadcasted_iota(jnp.int32, (BR, NG), 1)
        mf = (bt_ref[...] == ids).astype(jnp.float32)
        s_ref[...] += lax.dot_general(mf, h4, (((0,), (0,)), ((), ())),
                                      preferred_element_type=jnp.float32)
        onec = jnp.ones((BR, 1), jnp.float32)
        c_ref[...] += lax.dot_general(mf, onec, (((0,), (0,)), ((), ())),
                                      preferred_element_type=jnp.float32)
        rows = [jnp.max(mf[:, g0:g0 + 1] * h4, axis=0, keepdims=True)
                for g0 in range(NG)]
        cur = jnp.concatenate(rows, axis=0)
        x_ref[...] = jnp.maximum(x_ref[...], cur)

    part_spec = pl.BlockSpec((NC, BR, HID), lambda i: (0, i, 0))
    return pl.pallas_call(
        body,
        grid=(GRID,),
        in_specs=[
            part_spec, part_spec, part_spec, part_spec,
            pl.BlockSpec((BR, NHEAD), lambda i: (i, 0)),
            pl.BlockSpec((BR, HID), lambda i: (i, 0)),
            pl.BlockSpec((1, HID), lambda i: (0, 0)),
            pl.BlockSpec((BR, 1), lambda i: (i, 0)),
        ],
        out_specs=[
            pl.BlockSpec((NG, HID), lambda i: (0, 0)),
            pl.BlockSpec((NG, HID), lambda i: (0, 0)),
            pl.BlockSpec((NG, 1), lambda i: (0, 0)),
        ],
        out_shape=[
            jax.ShapeDtypeStruct((NG, HID), jnp.float32),
            jax.ShapeDtypeStruct((NG, HID), jnp.float32),
            jax.ShapeDtypeStruct((NG, 1), jnp.float32),
        ],
    )(b0, b1, b2, b3, rw, msg_self, b, batch2)


def _tc_classifier(sp, xp, cnt, W1, b1, W2, b2, W3, b3):
    def body(s_ref, x_ref, c_ref, w1_ref, b1_ref, w2_ref, b2_ref,
             w3_ref, b3_ref, o_ref):
        cnt = c_ref[...]
        mean = s_ref[...] / jnp.maximum(cnt, 1.0)
        mx = jnp.where(cnt > 0.0, x_ref[...], 0.0)
        rep = jnp.concatenate([mean, mx], axis=1)
        z = jnp.maximum(jnp.dot(rep, w1_ref[...],
                                preferred_element_type=jnp.float32)
                        + b1_ref[...], 0.0)
        z = jnp.maximum(jnp.dot(z, w2_ref[...],
                                preferred_element_type=jnp.float32)
                        + b2_ref[...], 0.0)
        lg = jnp.dot(z, w3_ref[...], preferred_element_type=jnp.float32) \
            + b3_ref[...]
        m = jnp.max(lg, axis=1, keepdims=True)
        e = jnp.exp(lg - m)
        o_ref[...] = e / jnp.sum(e, axis=1, keepdims=True)

    return pl.pallas_call(
        body,
        out_shape=jax.ShapeDtypeStruct((NG, 3), jnp.float32),
    )(sp, xp, cnt, W1, b1, W2, b2, W3, b3)


def kernel(x, edge_index, batch, W_gcn1, b_gcn1, W_gcn2, b_gcn2, W_gcn3,
           b_gcn3, W_gat, a_src, a_dst, b_gat, W_c1, b_c1, W_c2, b_c2,
           W_c3, b_c3):
    src3 = edge_index[0]
    dst3 = edge_index[1]
    z128 = jnp.zeros((N, HID), jnp.float32)
    ones128 = jnp.ones((CH, HID), jnp.float32)
    batch2 = batch.reshape(N, 1)

    deg_part = _sc_degree(dst3, z128, ones128)
    Y1, dinv = _tc_first(deg_part, x, W_gcn1)
    P1 = _sc_edge_agg(Y1, src3, dst3, z128)
    Y2 = _tc_combine_mm(P1, Y1, dinv, b_gcn1.reshape(1, HID), W_gcn2)
    P2 = _sc_edge_agg(Y2, src3, dst3, z128)
    Y3 = _tc_combine_mm(P2, Y2, dinv, b_gcn2.reshape(1, HID), W_gcn3)
    P3 = _sc_edge_agg(Y3, src3, dst3, z128)
    g, als, ald, exl = _tc_gat_proj(P3, Y3, dinv, b_gcn3.reshape(1, HID),
                                    W_gat, a_src.reshape(1, GDIM),
                                    a_dst.reshape(1, GDIM))
    ex_e, den_part = _sc_gat_logits(als, ald, src3, dst3, z128)
    rw, msg_self = _tc_rden(den_part, exl, g)
    g4 = g.reshape(N * NHEAD, HID)
    bs = [_sc_gat_heads[h](g4, ex_e, src3, dst3, z128) for h in range(NHEAD)]
    sp, xp, cnt = _tc_pool(bs[0], bs[1], bs[2], bs[3], rw, msg_self,
                           b_gat.reshape(1, HID), batch2)
    return _tc_classifier(sp, xp, cnt, W_c1, b_c1.reshape(1, HID),
                          W_c2, b_c2.reshape(1, HID // 2),
                          W_c3, b_c3.reshape(1, 3))
